# baseline (speedup 1.0000x reference)
import functools

import jax
import jax.numpy as jnp
from jax.experimental import pallas as pl
from jax.experimental.pallas import tpu as pltpu

_T = 8
_F = 3
_HD = 64


def _gclstm_kernel(x_ref, w_all_ref, wcomb_ref, par_ref, out_ref,
                   s_ref, *, n_valid, n_blocks, blk):
    i = pl.program_id(0)

    s_ref[_F:8, :] = jnp.concatenate(
        [jnp.ones((1, blk), jnp.float32),
         jnp.zeros((8 - _F - 1, blk), jnp.float32)], axis=0)

    w_all = w_all_ref[...]
    wcomb = wcomb_ref[...]
    par = par_ref[...]
    wci = par[:, 0:1]
    wcf = par[:, 1:2]
    wco = par[:, 2:3]

    c = jnp.zeros((_HD, blk), jnp.float32)
    h2 = None
    for t in range(_T):
        s_ref[0:_F, :] = x_ref[_F * t:_F * t + _F, :]
        if t > 0:
            mm = jnp.dot(wcomb, s_ref[...], preferred_element_type=jnp.float32)
        else:
            mm = jnp.dot(w_all, s_ref[0:8, :],
                         preferred_element_type=jnp.float32)
        gt = jnp.tanh(mm[2 * _HD:3 * _HD])
        if t > 0:
            ti = jnp.tanh(mm[0:_HD] + wci * c)
            tf = jnp.tanh(mm[_HD:2 * _HD] + wcf * c)
            c = 0.5 * (tf * c + c + ti * gt + gt)
        else:
            ti = jnp.tanh(mm[0:_HD])
            c = 0.5 * (ti * gt + gt)
        to = jnp.tanh(mm[3 * _HD:4 * _HD] + wco * c)
        tc = jnp.tanh(c)
        h2 = to * tc + tc
        if t < _T - 1:
            s_ref[8:8 + _HD, :] = h2

    @pl.when(i < n_blocks - 1)
    def _():
        out_ref[...] = jnp.sum(h2, axis=1).reshape(1, 1, _HD)

    @pl.when(i == n_blocks - 1)
    def _():
        lane = jax.lax.broadcasted_iota(jnp.int32, (_HD, blk), 1)
        rem = n_valid - (n_blocks - 1) * blk
        hm = jnp.where(lane < rem, h2, 0.0)
        out_ref[...] = jnp.sum(hm, axis=1).reshape(1, 1, _HD)


def kernel(x_seq, edge_index, W_i, W_f, W_c, W_o, Theta_i, Theta_f, Theta_c,
           Theta_o, bc_i, bc_f, bc_c, bc_o, w_c_i, w_c_f, w_c_o, b_i, b_f,
           b_c, b_o, W_lin, b_lin):
    del edge_index
    T, N, F = x_seq.shape
    blk = 8192
    n_blocks = pl.cdiv(N, blk)

    x2 = x_seq.transpose(0, 2, 1).reshape(T * F, N)

    wx = jnp.concatenate(
        [0.5 * W_i, 0.5 * W_f, W_c, 0.5 * W_o], axis=1).T
    bias = jnp.concatenate(
        [0.5 * (bc_i + b_i[0]), 0.5 * (bc_f + b_f[0]), bc_c + b_c[0],
         0.5 * (bc_o + b_o[0])])[:, None]
    th_all = jnp.concatenate(
        [0.25 * Theta_i, 0.25 * Theta_f, 0.5 * Theta_c, 0.25 * Theta_o],
        axis=1).T
    w_all = jnp.concatenate(
        [wx, bias, jnp.zeros((4 * _HD, 4), jnp.float32)], axis=1)
    wcomb = jnp.concatenate([w_all, th_all], axis=1)
    par = jnp.stack([0.5 * w_c_i[0], 0.5 * w_c_f[0], 0.5 * w_c_o[0],
                     jnp.zeros((_HD,), jnp.float32)], axis=1)

    partials = pl.pallas_call(
        functools.partial(_gclstm_kernel, n_valid=N, n_blocks=n_blocks,
                          blk=blk),
        grid=(n_blocks,),
        in_specs=[
            pl.BlockSpec((T * F, blk), lambda i: (0, i)),
            pl.BlockSpec((4 * _HD, 8), lambda i: (0, 0)),
            pl.BlockSpec((4 * _HD, 8 + _HD), lambda i: (0, 0)),
            pl.BlockSpec((_HD, 4), lambda i: (0, 0)),
        ],
        out_specs=pl.BlockSpec((1, 1, _HD), lambda i: (i, 0, 0)),
        out_shape=jax.ShapeDtypeStruct((n_blocks, 1, _HD), jnp.float32),
        scratch_shapes=[pltpu.VMEM((8 + _HD, blk), jnp.float32)],
        compiler_params=pltpu.CompilerParams(
            dimension_semantics=("parallel",)),
    )(x2, w_all, wcomb, par)
    g = jnp.sum(partials[:, 0, :], axis=0) * (0.5 / N)
    return g[None, :] @ W_lin + b_lin


# speedup vs baseline: 1.0885x; 1.0885x over previous
import functools

import jax
import jax.numpy as jnp
from jax.experimental import pallas as pl
from jax.experimental.pallas import tpu as pltpu

_T = 8
_F = 3
_HD = 64


def _gclstm_kernel(x_ref, w_all_ref, wcomb_ref, par_ref, out_ref,
                   s_ref, *, n_valid, n_blocks, blk):
    i = pl.program_id(0)

    s_ref[_F:8, :] = jnp.concatenate(
        [jnp.ones((1, blk), jnp.float32),
         jnp.zeros((8 - _F - 1, blk), jnp.float32)], axis=0)

    w_all = w_all_ref[...]
    wcomb = wcomb_ref[...]
    par = par_ref[...]
    wci = par[:, 0:1]
    wcf = par[:, 1:2]
    wco = par[:, 2:3]

    c = jnp.zeros((_HD, blk), jnp.float32)
    h2 = None
    for t in range(_T):
        s_ref[0:_F, :] = x_ref[_F * t:_F * t + _F, :]
        if t > 0:
            mm = jnp.dot(wcomb, s_ref[...], preferred_element_type=jnp.float32)
        else:
            mm = jnp.dot(w_all, s_ref[0:8, :],
                         preferred_element_type=jnp.float32)
        gt = jnp.tanh(mm[2 * _HD:3 * _HD])
        if t > 0:
            ti = jnp.tanh(mm[0:_HD] + wci * c)
            tf = jnp.tanh(mm[_HD:2 * _HD] + wcf * c)
            c = 0.5 * (tf * c + c + ti * gt + gt)
        else:
            ti = jnp.tanh(mm[0:_HD])
            c = 0.5 * (ti * gt + gt)
        to = jnp.tanh(mm[3 * _HD:4 * _HD] + wco * c)
        tc = jnp.tanh(c)
        h2 = to * tc + tc
        if t < _T - 1:
            s_ref[8:8 + _HD, :] = h2

    @pl.when(i < n_blocks - 1)
    def _():
        out_ref[...] = jnp.sum(h2, axis=1).reshape(1, 1, _HD)

    @pl.when(i == n_blocks - 1)
    def _():
        lane = jax.lax.broadcasted_iota(jnp.int32, (_HD, blk), 1)
        rem = n_valid - (n_blocks - 1) * blk
        hm = jnp.where(lane < rem, h2, 0.0)
        out_ref[...] = jnp.sum(hm, axis=1).reshape(1, 1, _HD)


def kernel(x_seq, edge_index, W_i, W_f, W_c, W_o, Theta_i, Theta_f, Theta_c,
           Theta_o, bc_i, bc_f, bc_c, bc_o, w_c_i, w_c_f, w_c_o, b_i, b_f,
           b_c, b_o, W_lin, b_lin):
    del edge_index
    T, N, F = x_seq.shape
    blk = 5120
    n_blocks = pl.cdiv(N, blk)

    x2 = x_seq.transpose(0, 2, 1).reshape(T * F, N)

    wx = jnp.concatenate(
        [0.5 * W_i, 0.5 * W_f, W_c, 0.5 * W_o], axis=1).T
    bias = jnp.concatenate(
        [0.5 * (bc_i + b_i[0]), 0.5 * (bc_f + b_f[0]), bc_c + b_c[0],
         0.5 * (bc_o + b_o[0])])[:, None]
    th_all = jnp.concatenate(
        [0.25 * Theta_i, 0.25 * Theta_f, 0.5 * Theta_c, 0.25 * Theta_o],
        axis=1).T
    w_all = jnp.concatenate(
        [wx, bias, jnp.zeros((4 * _HD, 4), jnp.float32)], axis=1)
    wcomb = jnp.concatenate([w_all, th_all], axis=1)
    par = jnp.stack([0.5 * w_c_i[0], 0.5 * w_c_f[0], 0.5 * w_c_o[0],
                     jnp.zeros((_HD,), jnp.float32)], axis=1)

    partials = pl.pallas_call(
        functools.partial(_gclstm_kernel, n_valid=N, n_blocks=n_blocks,
                          blk=blk),
        grid=(n_blocks,),
        in_specs=[
            pl.BlockSpec((T * F, blk), lambda i: (0, i)),
            pl.BlockSpec((4 * _HD, 8), lambda i: (0, 0)),
            pl.BlockSpec((4 * _HD, 8 + _HD), lambda i: (0, 0)),
            pl.BlockSpec((_HD, 4), lambda i: (0, 0)),
        ],
        out_specs=pl.BlockSpec((1, 1, _HD), lambda i: (i, 0, 0)),
        out_shape=jax.ShapeDtypeStruct((n_blocks, 1, _HD), jnp.float32),
        scratch_shapes=[pltpu.VMEM((8 + _HD, blk), jnp.float32)],
        compiler_params=pltpu.CompilerParams(
            dimension_semantics=("parallel",)),
    )(x2, w_all, wcomb, par)
    g = jnp.sum(partials[:, 0, :], axis=0) * (0.5 / N)
    return g[None, :] @ W_lin + b_lin


# speedup vs baseline: 1.0973x; 1.0081x over previous
import functools

import jax
import jax.numpy as jnp
from jax.experimental import pallas as pl
from jax.experimental.pallas import tpu as pltpu

_T = 8
_F = 3
_HD = 64


def _gclstm_kernel(x_ref, w_all_ref, wcomb_ref, par_ref, out_ref,
                   s_ref, *, n_valid, n_blocks, blk):
    i = pl.program_id(0)

    s_ref[_F:8, :] = jnp.concatenate(
        [jnp.ones((1, blk), jnp.float32),
         jnp.zeros((8 - _F - 1, blk), jnp.float32)], axis=0)

    w_all = w_all_ref[...]
    wcomb = wcomb_ref[...]
    par = par_ref[...]
    wci = par[:, 0:1]
    wcf = par[:, 1:2]
    wco = par[:, 2:3]

    c = jnp.zeros((_HD, blk), jnp.float32)
    h2 = None
    for t in range(_T):
        s_ref[0:_F, :] = x_ref[_F * t:_F * t + _F, :]
        if t > 0:
            mm = jnp.dot(wcomb, s_ref[...], preferred_element_type=jnp.float32)
        else:
            mm = jnp.dot(w_all, s_ref[0:8, :],
                         preferred_element_type=jnp.float32)
        gt = jnp.tanh(mm[2 * _HD:3 * _HD])
        if t > 0:
            ti = jnp.tanh(mm[0:_HD] + wci * c)
            tf = jnp.tanh(mm[_HD:2 * _HD] + wcf * c)
            c = 0.5 * (tf * c + c + ti * gt + gt)
        else:
            ti = jnp.tanh(mm[0:_HD])
            c = 0.5 * (ti * gt + gt)
        to = jnp.tanh(mm[3 * _HD:4 * _HD] + wco * c)
        tc = jnp.tanh(c)
        h2 = to * tc + tc
        if t < _T - 1:
            s_ref[8:8 + _HD, :] = h2

    @pl.when(i < n_blocks - 1)
    def _():
        out_ref[...] = jnp.sum(h2, axis=1).reshape(1, 1, _HD)

    @pl.when(i == n_blocks - 1)
    def _():
        lane = jax.lax.broadcasted_iota(jnp.int32, (_HD, blk), 1)
        rem = n_valid - (n_blocks - 1) * blk
        hm = jnp.where(lane < rem, h2, 0.0)
        out_ref[...] = jnp.sum(hm, axis=1).reshape(1, 1, _HD)


def kernel(x_seq, edge_index, W_i, W_f, W_c, W_o, Theta_i, Theta_f, Theta_c,
           Theta_o, bc_i, bc_f, bc_c, bc_o, w_c_i, w_c_f, w_c_o, b_i, b_f,
           b_c, b_o, W_lin, b_lin):
    del edge_index
    T, N, F = x_seq.shape
    blk = 6400
    n_blocks = pl.cdiv(N, blk)

    x2 = x_seq.transpose(0, 2, 1).reshape(T * F, N)

    wx = jnp.concatenate(
        [0.5 * W_i, 0.5 * W_f, W_c, 0.5 * W_o], axis=1).T
    bias = jnp.concatenate(
        [0.5 * (bc_i + b_i[0]), 0.5 * (bc_f + b_f[0]), bc_c + b_c[0],
         0.5 * (bc_o + b_o[0])])[:, None]
    th_all = jnp.concatenate(
        [0.25 * Theta_i, 0.25 * Theta_f, 0.5 * Theta_c, 0.25 * Theta_o],
        axis=1).T
    w_all = jnp.concatenate(
        [wx, bias, jnp.zeros((4 * _HD, 4), jnp.float32)], axis=1)
    wcomb = jnp.concatenate([w_all, th_all], axis=1)
    par = jnp.stack([0.5 * w_c_i[0], 0.5 * w_c_f[0], 0.5 * w_c_o[0],
                     jnp.zeros((_HD,), jnp.float32)], axis=1)

    partials = pl.pallas_call(
        functools.partial(_gclstm_kernel, n_valid=N, n_blocks=n_blocks,
                          blk=blk),
        grid=(n_blocks,),
        in_specs=[
            pl.BlockSpec((T * F, blk), lambda i: (0, i)),
            pl.BlockSpec((4 * _HD, 8), lambda i: (0, 0)),
            pl.BlockSpec((4 * _HD, 8 + _HD), lambda i: (0, 0)),
            pl.BlockSpec((_HD, 4), lambda i: (0, 0)),
        ],
        out_specs=pl.BlockSpec((1, 1, _HD), lambda i: (i, 0, 0)),
        out_shape=jax.ShapeDtypeStruct((n_blocks, 1, _HD), jnp.float32),
        scratch_shapes=[pltpu.VMEM((8 + _HD, blk), jnp.float32)],
        compiler_params=pltpu.CompilerParams(
            dimension_semantics=("parallel",)),
    )(x2, w_all, wcomb, par)
    g = jnp.sum(partials[:, 0, :], axis=0) * (0.5 / N)
    return g[None, :] @ W_lin + b_lin


# speedup vs baseline: 1.1190x; 1.0197x over previous
import functools

import jax
import jax.numpy as jnp
from jax.experimental import pallas as pl
from jax.experimental.pallas import tpu as pltpu

_T = 8
_F = 3
_HD = 64


def _gclstm_kernel(x_ref, w_all_ref, wcomb_ref, par_ref, out_ref,
                   s_ref, *, n_valid, n_blocks, blk):
    i = pl.program_id(0)

    s_ref[_F:8, :] = jnp.concatenate(
        [jnp.ones((1, blk), jnp.float32),
         jnp.zeros((8 - _F - 1, blk), jnp.float32)], axis=0)

    w_all = w_all_ref[...]
    wcomb = wcomb_ref[...]
    par = par_ref[...]
    wci = par[:, 0:1]
    wcf = par[:, 1:2]
    wco = par[:, 2:3]

    c = jnp.zeros((_HD, blk), jnp.float32)
    h2 = None
    for t in range(_T):
        s_ref[0:_F, :] = x_ref[_F * t:_F * t + _F, :]
        if t > 0:
            mm = jnp.dot(wcomb, s_ref[...], preferred_element_type=jnp.float32)
        else:
            mm = jnp.dot(w_all, s_ref[0:8, :],
                         preferred_element_type=jnp.float32)
        gt = jnp.tanh(mm[2 * _HD:3 * _HD])
        if t > 0:
            ti = jnp.tanh(mm[0:_HD] + wci * c)
            tf = jnp.tanh(mm[_HD:2 * _HD] + wcf * c)
            c = 0.5 * (tf * c + c + ti * gt + gt)
        else:
            ti = jnp.tanh(mm[0:_HD])
            c = 0.5 * (ti * gt + gt)
        to = jnp.tanh(mm[3 * _HD:4 * _HD] + wco * c)
        tc = jnp.tanh(c)
        h2 = to * tc + tc
        if t < _T - 1:
            s_ref[8:8 + _HD, :] = h2

    @pl.when(i < n_blocks - 1)
    def _():
        out_ref[...] = jnp.sum(h2, axis=1).reshape(1, 1, _HD)

    @pl.when(i == n_blocks - 1)
    def _():
        lane = jax.lax.broadcasted_iota(jnp.int32, (_HD, blk), 1)
        rem = n_valid - (n_blocks - 1) * blk
        hm = jnp.where(lane < rem, h2, 0.0)
        out_ref[...] = jnp.sum(hm, axis=1).reshape(1, 1, _HD)


def kernel(x_seq, edge_index, W_i, W_f, W_c, W_o, Theta_i, Theta_f, Theta_c,
           Theta_o, bc_i, bc_f, bc_c, bc_o, w_c_i, w_c_f, w_c_o, b_i, b_f,
           b_c, b_o, W_lin, b_lin):
    del edge_index
    T, N, F = x_seq.shape
    blk = 7168
    n_blocks = pl.cdiv(N, blk)

    x2 = x_seq.transpose(0, 2, 1).reshape(T * F, N)

    wx = jnp.concatenate(
        [0.5 * W_i, 0.5 * W_f, W_c, 0.5 * W_o], axis=1).T
    bias = jnp.concatenate(
        [0.5 * (bc_i + b_i[0]), 0.5 * (bc_f + b_f[0]), bc_c + b_c[0],
         0.5 * (bc_o + b_o[0])])[:, None]
    th_all = jnp.concatenate(
        [0.25 * Theta_i, 0.25 * Theta_f, 0.5 * Theta_c, 0.25 * Theta_o],
        axis=1).T
    w_all = jnp.concatenate(
        [wx, bias, jnp.zeros((4 * _HD, 4), jnp.float32)], axis=1)
    wcomb = jnp.concatenate([w_all, th_all], axis=1)
    par = jnp.stack([0.5 * w_c_i[0], 0.5 * w_c_f[0], 0.5 * w_c_o[0],
                     jnp.zeros((_HD,), jnp.float32)], axis=1)

    partials = pl.pallas_call(
        functools.partial(_gclstm_kernel, n_valid=N, n_blocks=n_blocks,
                          blk=blk),
        grid=(n_blocks,),
        in_specs=[
            pl.BlockSpec((T * F, blk), lambda i: (0, i)),
            pl.BlockSpec((4 * _HD, 8), lambda i: (0, 0)),
            pl.BlockSpec((4 * _HD, 8 + _HD), lambda i: (0, 0)),
            pl.BlockSpec((_HD, 4), lambda i: (0, 0)),
        ],
        out_specs=pl.BlockSpec((1, 1, _HD), lambda i: (i, 0, 0)),
        out_shape=jax.ShapeDtypeStruct((n_blocks, 1, _HD), jnp.float32),
        scratch_shapes=[pltpu.VMEM((8 + _HD, blk), jnp.float32)],
        compiler_params=pltpu.CompilerParams(
            dimension_semantics=("parallel",)),
    )(x2, w_all, wcomb, par)
    g = jnp.sum(partials[:, 0, :], axis=0) * (0.5 / N)
    return g[None, :] @ W_lin + b_lin
